# Initial kernel scaffold; baseline (speedup 1.0000x reference)
import jax
import jax.numpy as jnp
from jax.experimental import pallas as pl


def kernel(inputs, embeddings):
    raise NotImplementedError("write your pallas kernel here")


# speedup vs baseline: 1.5215x; 1.5215x over previous
import functools

import jax
import jax.numpy as jnp
from jax import lax
from jax.experimental import pallas as pl
from jax.experimental.pallas import tpu as pltpu
from jax.experimental.pallas import tpu_sc as plsc

CHUNK = 128


@functools.partial(jax.jit, static_argnums=(2, 3, 4))
def _sc_gather(idx2, table, chunks_per_w, num_workers, embed_dim):
    n = idx2.shape[0] * CHUNK
    mesh = plsc.VectorSubcoreMesh(core_axis_name="c", subcore_axis_name="s")
    nc = mesh.num_cores

    @functools.partial(
        pl.kernel,
        out_type=jax.ShapeDtypeStruct((n, embed_dim), jnp.float32),
        mesh=mesh,
        scratch_types=[
            pltpu.VMEM((chunks_per_w, CHUNK), jnp.int32),
            pltpu.VMEM((CHUNK, embed_dim), jnp.float32),
            pltpu.VMEM((CHUNK, embed_dim), jnp.float32),
            pltpu.SemaphoreType.DMA,
            pltpu.SemaphoreType.DMA,
            pltpu.SemaphoreType.DMA,
            pltpu.SemaphoreType.DMA,
        ],
        compiler_params=pltpu.CompilerParams(use_tc_tiling_on_sc=False),
    )
    def run(idx_hbm, table_hbm, out_hbm, idxv, rows0, rows1, g0, g1, o0, o1):
        wid = lax.axis_index("s") * nc + lax.axis_index("c")
        cbase = wid * chunks_per_w
        rbase = cbase * CHUNK
        pltpu.sync_copy(idx_hbm.at[pl.ds(cbase, chunks_per_w)], idxv)

        rows = (rows0, rows1)
        gsem = (g0, g1)
        osem = (o0, o1)

        def start_gather(c, b):
            pltpu.async_copy(table_hbm.at[idxv.at[c]], rows[b], gsem[b])

        def wait_gather(c, b):
            pltpu.make_async_copy(table_hbm.at[idxv.at[c]], rows[b], gsem[b]).wait()

        def start_out(c, b):
            pltpu.async_copy(
                rows[b], out_hbm.at[pl.ds(rbase + c * CHUNK, CHUNK)], osem[b]
            )

        def wait_out(c, b):
            pltpu.make_async_copy(
                rows[b], out_hbm.at[pl.ds(rbase + c * CHUNK, CHUNK)], osem[b]
            ).wait()

        start_gather(0, 0)
        wait_gather(0, 0)
        start_out(0, 0)
        start_gather(1, 1)

        def mid(i, _):
            for j, b in ((0, 1), (1, 0)):
                c = 1 + 2 * i + j
                nb = 1 - b
                wait_out(c - 1, nb)
                start_gather(c + 1, nb)
                wait_gather(c, b)
                start_out(c, b)
            return 0

        lax.fori_loop(0, (chunks_per_w - 2) // 2, mid, 0)

        last = chunks_per_w - 1
        wait_gather(last, 1)
        start_out(last, 1)
        wait_out(last - 1, 0)
        wait_out(last, 1)

    return run(idx2, table)


def kernel(inputs, embeddings):
    b, f = inputs.shape
    v, d = embeddings.shape
    n = b * f
    num_workers = 32
    assert n % (num_workers * CHUNK) == 0
    chunks_per_w = n // (num_workers * CHUNK)
    idx2 = inputs.astype(jnp.int32).reshape(n // CHUNK, CHUNK)
    out = _sc_gather(idx2, embeddings, chunks_per_w, num_workers, d)
    return out.reshape(b, f, d)


# speedup vs baseline: 1.5747x; 1.0350x over previous
import functools

import jax
import jax.numpy as jnp
from jax import lax
from jax.experimental import pallas as pl
from jax.experimental.pallas import tpu as pltpu
from jax.experimental.pallas import tpu_sc as plsc

CHUNK = 512


@functools.partial(jax.jit, static_argnums=(2, 3, 4))
def _sc_gather(idx2, table, chunks_per_w, num_workers, embed_dim):
    n = idx2.shape[0] * CHUNK
    mesh = plsc.VectorSubcoreMesh(core_axis_name="c", subcore_axis_name="s")
    nc = mesh.num_cores

    @functools.partial(
        pl.kernel,
        out_type=jax.ShapeDtypeStruct((n, embed_dim), jnp.float32),
        mesh=mesh,
        scratch_types=[
            pltpu.VMEM((chunks_per_w, CHUNK), jnp.int32),
            pltpu.VMEM((CHUNK, embed_dim), jnp.float32),
            pltpu.VMEM((CHUNK, embed_dim), jnp.float32),
            pltpu.SemaphoreType.DMA,
            pltpu.SemaphoreType.DMA,
            pltpu.SemaphoreType.DMA,
            pltpu.SemaphoreType.DMA,
        ],
        compiler_params=pltpu.CompilerParams(use_tc_tiling_on_sc=False),
    )
    def run(idx_hbm, table_hbm, out_hbm, idxv, rows0, rows1, g0, g1, o0, o1):
        wid = lax.axis_index("s") * nc + lax.axis_index("c")
        cbase = wid * chunks_per_w
        rbase = cbase * CHUNK
        pltpu.sync_copy(idx_hbm.at[pl.ds(cbase, chunks_per_w)], idxv)

        rows = (rows0, rows1)
        gsem = (g0, g1)
        osem = (o0, o1)

        def start_gather(c, b):
            pltpu.async_copy(table_hbm.at[idxv.at[c]], rows[b], gsem[b])

        def wait_gather(c, b):
            pltpu.make_async_copy(table_hbm.at[idxv.at[c]], rows[b], gsem[b]).wait()

        def start_out(c, b):
            pltpu.async_copy(
                rows[b], out_hbm.at[pl.ds(rbase + c * CHUNK, CHUNK)], osem[b]
            )

        def wait_out(c, b):
            pltpu.make_async_copy(
                rows[b], out_hbm.at[pl.ds(rbase + c * CHUNK, CHUNK)], osem[b]
            ).wait()

        start_gather(0, 0)
        wait_gather(0, 0)
        start_out(0, 0)
        start_gather(1, 1)

        def mid(i, _):
            for j, b in ((0, 1), (1, 0)):
                c = 1 + 2 * i + j
                nb = 1 - b
                wait_out(c - 1, nb)
                start_gather(c + 1, nb)
                wait_gather(c, b)
                start_out(c, b)
            return 0

        lax.fori_loop(0, (chunks_per_w - 2) // 2, mid, 0)

        last = chunks_per_w - 1
        wait_gather(last, 1)
        start_out(last, 1)
        wait_out(last - 1, 0)
        wait_out(last, 1)

    return run(idx2, table)


def kernel(inputs, embeddings):
    b, f = inputs.shape
    v, d = embeddings.shape
    n = b * f
    num_workers = 32
    assert n % (num_workers * CHUNK) == 0
    chunks_per_w = n // (num_workers * CHUNK)
    idx2 = inputs.astype(jnp.int32).reshape(n // CHUNK, CHUNK)
    out = _sc_gather(idx2, embeddings, chunks_per_w, num_workers, d)
    return out.reshape(b, f, d)


# speedup vs baseline: 1.5780x; 1.0020x over previous
import functools

import jax
import jax.numpy as jnp
from jax import lax
from jax.experimental import pallas as pl
from jax.experimental.pallas import tpu as pltpu
from jax.experimental.pallas import tpu_sc as plsc

CHUNK = 256
NBUF = 13
DEPTH = 8


@functools.partial(jax.jit, static_argnums=(2, 3, 4))
def _sc_gather(idx2, table, nch, num_workers, embed_dim):
    n = idx2.shape[0] * CHUNK
    mesh = plsc.VectorSubcoreMesh(core_axis_name="c", subcore_axis_name="s")
    nc = mesh.num_cores
    slack = NBUF - DEPTH
    assert nch % NBUF == 0 and 0 < slack < nch - DEPTH

    @functools.partial(
        pl.kernel,
        out_type=jax.ShapeDtypeStruct((n, embed_dim), jnp.float32),
        mesh=mesh,
        scratch_types=(
            [pltpu.VMEM((nch, CHUNK), jnp.int32)]
            + [pltpu.VMEM((NBUF * CHUNK, embed_dim), jnp.float32)]
            + [pltpu.SemaphoreType.DMA] * (2 * NBUF)
        ),
        compiler_params=pltpu.CompilerParams(use_tc_tiling_on_sc=False),
    )
    def run(idx_hbm, table_hbm, out_hbm, idxv, rowsv, *sems):
        gsem = sems[:NBUF]
        osem = sems[NBUF:]
        wid = lax.axis_index("s") * nc + lax.axis_index("c")
        cbase = wid * nch
        rbase = cbase * CHUNK
        pltpu.sync_copy(idx_hbm.at[pl.ds(cbase, nch)], idxv)

        def rows(b):
            return rowsv.at[pl.ds(b * CHUNK, CHUNK)]

        def start_gather(c, b):
            pltpu.async_copy(table_hbm.at[idxv.at[c]], rows(b), gsem[b])

        def wait_gather(c, b):
            pltpu.make_async_copy(
                table_hbm.at[idxv.at[c]], rows(b), gsem[b]
            ).wait()

        def start_out(c, b):
            pltpu.async_copy(
                rows(b), out_hbm.at[pl.ds(rbase + c * CHUNK, CHUNK)], osem[b]
            )

        def wait_out(c, b):
            pltpu.make_async_copy(
                rows(b), out_hbm.at[pl.ds(rbase + c * CHUNK, CHUNK)], osem[b]
            ).wait()

        def step(c, bi):
            wait_out(c - slack, (bi - slack) % NBUF)
            start_gather(c + DEPTH, (bi - slack) % NBUF)
            wait_gather(c, bi)
            start_out(c, bi)

        for c in range(DEPTH):
            start_gather(c, c % NBUF)

        for c in range(slack):
            start_gather(c + DEPTH, (c + DEPTH) % NBUF)
            wait_gather(c, c % NBUF)
            start_out(c, c % NBUF)

        n_mid = (nch - NBUF) // NBUF

        def mid(i, _):
            for j in range(NBUF):
                c = slack + i * NBUF + j
                step(c, (slack + j) % NBUF)
            return 0

        lax.fori_loop(0, n_mid, mid, 0)

        for c in range(nch - DEPTH, nch):
            wait_out(c - slack, (c - slack) % NBUF)
            wait_gather(c, c % NBUF)
            start_out(c, c % NBUF)

        for c in range(nch - slack, nch):
            wait_out(c, c % NBUF)

    return run(idx2, table)


def kernel(inputs, embeddings):
    b, f = inputs.shape
    v, d = embeddings.shape
    n = b * f
    num_workers = 32
    assert n % (num_workers * CHUNK) == 0
    nch = n // (num_workers * CHUNK)
    idx2 = inputs.astype(jnp.int32).reshape(n // CHUNK, CHUNK)
    out = _sc_gather(idx2, embeddings, nch, num_workers, d)
    return out.reshape(b, f, d)
